# Initial kernel scaffold; baseline (speedup 1.0000x reference)
import jax
import jax.numpy as jnp
from jax.experimental import pallas as pl


def kernel(logits, labels):
    raise NotImplementedError("write your pallas kernel here")


# speedup vs baseline: 1.2690x; 1.2690x over previous
import functools

import jax
import jax.numpy as jnp
import numpy as np
from jax.experimental import pallas as pl
from jax.experimental.pallas import tpu as pltpu

N_BINS_K = 15
PAD_BINS = 16


def _ece_block_kernel(n_total, n_grid, x_ref, lab_ref, lo_ref, up_ref,
                      out_ref, acc_ref):
    i = pl.program_id(0)
    x = x_ref[...]
    m = jnp.max(x, axis=1, keepdims=True)
    s = jnp.sum(jnp.exp(x - m), axis=1, keepdims=True)
    conf = 1.0 / s
    c = x.shape[1]
    col = jax.lax.broadcasted_iota(jnp.int32, x.shape, 1)
    cand = jnp.where(x == m, col, c)
    pred = jnp.min(cand, axis=1)
    lab = lab_ref[0, 0, :]
    acc = (pred == lab).astype(jnp.float32)[:, None]

    lo = lo_ref[...]
    up = up_ref[...]
    inb = ((conf > lo) & (conf <= up)).astype(jnp.float32)
    cnt = jnp.sum(inb, axis=0, keepdims=True)
    sconf = jnp.sum(inb * conf, axis=0, keepdims=True)
    sacc = jnp.sum(inb * acc, axis=0, keepdims=True)
    upd = jnp.concatenate([cnt, sconf, sacc], axis=0)

    @pl.when(i == 0)
    def _init():
        acc_ref[...] = upd

    @pl.when(i > 0)
    def _accum():
        acc_ref[...] = acc_ref[...] + upd

    @pl.when(i == n_grid - 1)
    def _finish():
        tot = acc_ref[...]
        count = tot[0:1, :]
        tconf = tot[1:2, :]
        tacc = tot[2:3, :]
        denom = jnp.maximum(count, 1.0)
        contrib = jnp.abs(tconf / denom - tacc / denom) * (count / n_total)
        out_ref[...] = jnp.sum(jnp.where(count > 0.0, contrib, 0.0),
                               keepdims=True)


def kernel(logits, labels):
    n, c = logits.shape
    rows = 512
    grid = n // rows
    labels3 = labels.reshape(grid, 1, rows)

    bounds = np.linspace(0.0, 1.0, N_BINS_K + 1).astype(np.float32)
    lowers = np.full((1, PAD_BINS), 2.0, np.float32)
    uppers = np.full((1, PAD_BINS), 3.0, np.float32)
    lowers[0, :N_BINS_K] = bounds[:-1]
    uppers[0, :N_BINS_K] = bounds[1:]

    out = pl.pallas_call(
        functools.partial(_ece_block_kernel, float(n), grid),
        grid=(grid,),
        in_specs=[
            pl.BlockSpec((rows, c), lambda i: (i, 0)),
            pl.BlockSpec((1, 1, rows), lambda i: (i, 0, 0)),
            pl.BlockSpec((1, PAD_BINS), lambda i: (0, 0)),
            pl.BlockSpec((1, PAD_BINS), lambda i: (0, 0)),
        ],
        out_specs=pl.BlockSpec((1, 1), lambda i: (0, 0)),
        out_shape=jax.ShapeDtypeStruct((1, 1), jnp.float32),
        scratch_shapes=[pltpu.VMEM((3, PAD_BINS), jnp.float32)],
    )(logits, labels3, jnp.asarray(lowers), jnp.asarray(uppers))
    return out.reshape(1)
